# Initial kernel scaffold; baseline (speedup 1.0000x reference)
import jax
import jax.numpy as jnp
from jax.experimental import pallas as pl


def kernel(x, params, edge_index, batch):
    raise NotImplementedError("write your pallas kernel here")


# speedup vs baseline: 4.5042x; 4.5042x over previous
import functools

import jax
import jax.numpy as jnp
from jax import lax
from jax.experimental import pallas as pl
from jax.experimental.pallas import tpu as pltpu
from jax.experimental.pallas import tpu_sc as plsc

N = 10000
E = 320000
D = 128
H = 128
G = 64
OUT = 12
NUM_LAYERS = 3

NC = 2
NS = 16
EDGES_PER_CORE = E // NC
EDGES_PER_TILE = EDGES_PER_CORE // NS
CH = 80
NCHUNK = EDGES_PER_TILE // CH
ROWS_PER_TILE = 632

BN = 1000
NB = N // BN


_sc_mesh = plsc.VectorSubcoreMesh(core_axis_name="c", subcore_axis_name="s")


@functools.partial(
    pl.kernel,
    mesh=_sc_mesh,
    out_type=(
        jax.ShapeDtypeStruct((N, D), jnp.float32),
        jax.ShapeDtypeStruct((N, D), jnp.float32),
    ),
    scratch_types=[
        pltpu.VMEM((CH,), jnp.int32),
        pltpu.VMEM((CH,), jnp.int32),
        pltpu.VMEM((CH, D), jnp.float32),
        pltpu.VMEM_SHARED((N, D), jnp.float32),
        pltpu.SemaphoreType.DMA,
    ],
)
def _sc_agg(h_hbm, src_hbm, dst_hbm, zeros_hbm, out0, out1, src_v, dst_v,
            rows_v, acc_sh, sem):
    c = lax.axis_index("c")
    s = lax.axis_index("s")

    r0 = jnp.minimum(s * ROWS_PER_TILE, N - ROWS_PER_TILE)
    pltpu.sync_copy(zeros_hbm.at[pl.ds(r0, ROWS_PER_TILE)],
                    acc_sh.at[pl.ds(r0, ROWS_PER_TILE)])
    plsc.subcore_barrier()

    base = c * EDGES_PER_CORE + s * EDGES_PER_TILE

    def body(i, _):
        off = base + i * CH
        pltpu.sync_copy(src_hbm.at[pl.ds(off, CH)], src_v)
        pltpu.sync_copy(dst_hbm.at[pl.ds(off, CH)], dst_v)
        pltpu.async_copy(h_hbm.at[src_v], rows_v, sem).wait()
        pltpu.sync_copy(rows_v, acc_sh.at[dst_v], add=True)
        return 0

    lax.fori_loop(0, NCHUNK, body, 0)
    plsc.subcore_barrier()

    @pl.when(c == 0)
    def _():
        pltpu.sync_copy(acc_sh.at[pl.ds(r0, ROWS_PER_TILE)],
                        out0.at[pl.ds(r0, ROWS_PER_TILE)])

    @pl.when(c == 1)
    def _():
        pltpu.sync_copy(acc_sh.at[pl.ds(r0, ROWS_PER_TILE)],
                        out1.at[pl.ds(r0, ROWS_PER_TILE)])


def _gin_mlp_body(h_ref, a0_ref, a1_ref, w1_ref, b1_ref, w2_ref, b2_ref,
                  z_ref, st_ref):
    i = pl.program_id(0)
    h2 = h_ref[...] + a0_ref[...] + a1_ref[...]
    t = jnp.dot(h2, w1_ref[...], preferred_element_type=jnp.float32)
    t = jnp.maximum(t + b1_ref[...], 0.0)
    z = jnp.dot(t, w2_ref[...], preferred_element_type=jnp.float32)
    z = z + b2_ref[...]
    z_ref[...] = z
    st = jnp.concatenate(
        [jnp.sum(z, axis=0, keepdims=True),
         jnp.sum(z * z, axis=0, keepdims=True)], axis=0)

    @pl.when(i == 0)
    def _():
        st_ref[...] = st

    @pl.when(i > 0)
    def _():
        st_ref[...] = st_ref[...] + st


_gin_mlp = pl.pallas_call(
    _gin_mlp_body,
    grid=(NB,),
    in_specs=[
        pl.BlockSpec((BN, D), lambda i: (i, 0)),
        pl.BlockSpec((BN, D), lambda i: (i, 0)),
        pl.BlockSpec((BN, D), lambda i: (i, 0)),
        pl.BlockSpec((D, H), lambda i: (0, 0)),
        pl.BlockSpec((1, H), lambda i: (0, 0)),
        pl.BlockSpec((H, H), lambda i: (0, 0)),
        pl.BlockSpec((1, H), lambda i: (0, 0)),
    ],
    out_specs=[
        pl.BlockSpec((BN, H), lambda i: (i, 0)),
        pl.BlockSpec((2, H), lambda i: (0, 0)),
    ],
    out_shape=[
        jax.ShapeDtypeStruct((N, H), jnp.float32),
        jax.ShapeDtypeStruct((2, H), jnp.float32),
    ],
)


def _bn_relu_body(z_ref, st_ref, g_ref, be_ref, h_ref):
    mean = st_ref[0:1] * (1.0 / N)
    var = st_ref[1:2] * (1.0 / N) - mean * mean
    inv = lax.rsqrt(var + 1e-5)
    h_ref[...] = jnp.maximum(
        (z_ref[...] - mean) * (inv * g_ref[...]) + be_ref[...], 0.0)


_bn_relu = pl.pallas_call(
    _bn_relu_body,
    grid=(NB,),
    in_specs=[
        pl.BlockSpec((BN, H), lambda i: (i, 0)),
        pl.BlockSpec((2, H), lambda i: (0, 0)),
        pl.BlockSpec((1, H), lambda i: (0, 0)),
        pl.BlockSpec((1, H), lambda i: (0, 0)),
    ],
    out_specs=pl.BlockSpec((BN, H), lambda i: (i, 0)),
    out_shape=jax.ShapeDtypeStruct((N, H), jnp.float32),
)


def _pool_head_body(z_ref, st_ref, g_ref, be_ref, b_ref, hw1_ref, hb1_ref,
                    hw2_ref, hb2_ref, o_ref, acc_ref):
    i = pl.program_id(0)
    mean = st_ref[0:1] * (1.0 / N)
    var = st_ref[1:2] * (1.0 / N) - mean * mean
    inv = lax.rsqrt(var + 1e-5)
    h = jnp.maximum((z_ref[...] - mean) * (inv * g_ref[...]) + be_ref[...],
                    0.0)
    bvals = jnp.broadcast_to(b_ref[0], (G, BN))
    oh = (bvals == lax.broadcasted_iota(jnp.int32, (G, BN), 0)
          ).astype(jnp.float32)
    part = jnp.dot(oh, h, preferred_element_type=jnp.float32)

    @pl.when(i == 0)
    def _():
        acc_ref[...] = part

    @pl.when(i > 0)
    def _():
        acc_ref[...] = acc_ref[...] + part

    @pl.when(i == pl.num_programs(0) - 1)
    def _():
        gpool = acc_ref[...]
        a = jnp.dot(gpool, hw1_ref[...], preferred_element_type=jnp.float32)
        a = jnp.maximum(a + hb1_ref[...], 0.0)
        o = jnp.dot(a, hw2_ref[...], preferred_element_type=jnp.float32)
        o_ref[...] = jax.nn.sigmoid(o + hb2_ref[...])


_pool_head = pl.pallas_call(
    _pool_head_body,
    grid=(NB,),
    in_specs=[
        pl.BlockSpec((BN, H), lambda i: (i, 0)),
        pl.BlockSpec((2, H), lambda i: (0, 0)),
        pl.BlockSpec((1, H), lambda i: (0, 0)),
        pl.BlockSpec((1, H), lambda i: (0, 0)),
        pl.BlockSpec((1, 1, BN), lambda i: (i, 0, 0)),
        pl.BlockSpec((H, H), lambda i: (0, 0)),
        pl.BlockSpec((1, H), lambda i: (0, 0)),
        pl.BlockSpec((H, H), lambda i: (0, 0)),
        pl.BlockSpec((1, H), lambda i: (0, 0)),
    ],
    out_specs=pl.BlockSpec((G, H), lambda i: (0, 0)),
    out_shape=jax.ShapeDtypeStruct((G, H), jnp.float32),
    scratch_shapes=[pltpu.VMEM((G, H), jnp.float32)],
)


def kernel(x, params, edge_index, batch):
    src = edge_index[0]
    dst = edge_index[1]
    zeros = jnp.zeros((N, D), jnp.float32)
    batch3 = batch.reshape(NB, 1, BN)

    hw1p = jnp.zeros((H, H), jnp.float32).at[:, :H // 2].set(params["hw1"])
    hb1p = jnp.zeros((1, H), jnp.float32).at[0, :H // 2].set(params["hb1"])
    hw2p = jnp.zeros((H, H), jnp.float32).at[:H // 2, :OUT].set(params["hw2"])
    hb2p = jnp.zeros((1, H), jnp.float32).at[0, :OUT].set(params["hb2"])

    h = x
    z = st = None
    for i in range(NUM_LAYERS):
        a0, a1 = _sc_agg(h, src, dst, zeros)
        z, st = _gin_mlp(h, a0, a1,
                         params[f"w1_{i}"], params[f"b1_{i}"].reshape(1, H),
                         params[f"w2_{i}"], params[f"b2_{i}"].reshape(1, H))
        if i < NUM_LAYERS - 1:
            h = _bn_relu(z, st, params[f"gamma_{i}"].reshape(1, H),
                         params[f"beta_{i}"].reshape(1, H))

    out = _pool_head(z, st,
                     params[f"gamma_{NUM_LAYERS - 1}"].reshape(1, H),
                     params[f"beta_{NUM_LAYERS - 1}"].reshape(1, H),
                     batch3, hw1p, hb1p, hw2p, hb2p)
    return out[:, :OUT]


# speedup vs baseline: 8.1023x; 1.7988x over previous
import functools

import jax
import jax.numpy as jnp
from jax import lax
from jax.experimental import pallas as pl
from jax.experimental.pallas import tpu as pltpu
from jax.experimental.pallas import tpu_sc as plsc

N = 10000
E = 320000
D = 128
H = 128
G = 64
OUT = 12
NUM_LAYERS = 3

NC = 2
NS = 16
EDGES_PER_CORE = E // NC
EDGES_PER_TILE = EDGES_PER_CORE // NS
CH = 80
NCHUNK = EDGES_PER_TILE // CH
ROWS_PER_TILE = 632

BN = 1000
NB = N // BN


_sc_mesh = plsc.VectorSubcoreMesh(core_axis_name="c", subcore_axis_name="s")


@functools.partial(
    pl.kernel,
    mesh=_sc_mesh,
    out_type=(
        jax.ShapeDtypeStruct((N, D), jnp.float32),
        jax.ShapeDtypeStruct((N, D), jnp.float32),
    ),
    scratch_types=[
        pltpu.VMEM((EDGES_PER_TILE,), jnp.int32),
        pltpu.VMEM((EDGES_PER_TILE,), jnp.int32),
        pltpu.VMEM((CH,), jnp.int32),
        pltpu.VMEM((CH,), jnp.int32),
        pltpu.VMEM((CH, D), jnp.float32),
        pltpu.VMEM((CH, D), jnp.float32),
        pltpu.VMEM_SHARED((N, D), jnp.float32),
        pltpu.SemaphoreType.DMA,
        pltpu.SemaphoreType.DMA,
        pltpu.SemaphoreType.DMA,
        pltpu.SemaphoreType.DMA,
    ],
)
def _sc_agg(h_hbm, src_hbm, dst_hbm, zeros_hbm, out0, out1, srcf_v, dstf_v,
            di0_v, di1_v, rows0_v, rows1_v, acc_sh, sg0, sg1, ss0, ss1):
    c = lax.axis_index("c")
    s = lax.axis_index("s")

    base = c * EDGES_PER_CORE + s * EDGES_PER_TILE
    pltpu.sync_copy(src_hbm.at[pl.ds(base, EDGES_PER_TILE)], srcf_v)
    pltpu.sync_copy(dst_hbm.at[pl.ds(base, EDGES_PER_TILE)], dstf_v)

    r0 = jnp.minimum(s * ROWS_PER_TILE, N - ROWS_PER_TILE)
    pltpu.sync_copy(zeros_hbm.at[pl.ds(r0, ROWS_PER_TILE)],
                    acc_sh.at[pl.ds(r0, ROWS_PER_TILE)])
    plsc.subcore_barrier()

    def chunk_pair(k, _):
        o0 = 2 * k * CH
        o1 = o0 + CH
        g0 = pltpu.async_copy(h_hbm.at[srcf_v.at[pl.ds(o0, CH)]], rows0_v,
                              sg0)
        g1 = pltpu.async_copy(h_hbm.at[srcf_v.at[pl.ds(o1, CH)]], rows1_v,
                              sg1)
        g0.wait()
        s0 = pltpu.async_copy(rows0_v, acc_sh.at[dstf_v.at[pl.ds(o0, CH)]],
                              ss0, add=True)
        g1.wait()
        s1 = pltpu.async_copy(rows1_v, acc_sh.at[dstf_v.at[pl.ds(o1, CH)]],
                              ss1, add=True)
        s0.wait()
        s1.wait()
        return 0

    lax.fori_loop(0, NCHUNK // 2, chunk_pair, 0)
    ot = (NCHUNK - 1) * CH
    pltpu.async_copy(h_hbm.at[srcf_v.at[pl.ds(ot, CH)]], rows0_v, sg0).wait()
    pltpu.sync_copy(rows0_v, acc_sh.at[dstf_v.at[pl.ds(ot, CH)]], add=True)
    plsc.subcore_barrier()

    @pl.when(c == 0)
    def _():
        pltpu.sync_copy(acc_sh.at[pl.ds(r0, ROWS_PER_TILE)],
                        out0.at[pl.ds(r0, ROWS_PER_TILE)])

    @pl.when(c == 1)
    def _():
        pltpu.sync_copy(acc_sh.at[pl.ds(r0, ROWS_PER_TILE)],
                        out1.at[pl.ds(r0, ROWS_PER_TILE)])


def _gin_mlp_body(h_ref, a0_ref, a1_ref, w1_ref, b1_ref, w2_ref, b2_ref,
                  z_ref, st_ref):
    i = pl.program_id(0)
    h2 = h_ref[...] + a0_ref[...] + a1_ref[...]
    t = jnp.dot(h2, w1_ref[...], preferred_element_type=jnp.float32)
    t = jnp.maximum(t + b1_ref[...], 0.0)
    z = jnp.dot(t, w2_ref[...], preferred_element_type=jnp.float32)
    z = z + b2_ref[...]
    z_ref[...] = z
    st = jnp.concatenate(
        [jnp.sum(z, axis=0, keepdims=True),
         jnp.sum(z * z, axis=0, keepdims=True)], axis=0)

    @pl.when(i == 0)
    def _():
        st_ref[...] = st

    @pl.when(i > 0)
    def _():
        st_ref[...] = st_ref[...] + st


_gin_mlp = pl.pallas_call(
    _gin_mlp_body,
    grid=(NB,),
    in_specs=[
        pl.BlockSpec((BN, D), lambda i: (i, 0)),
        pl.BlockSpec((BN, D), lambda i: (i, 0)),
        pl.BlockSpec((BN, D), lambda i: (i, 0)),
        pl.BlockSpec((D, H), lambda i: (0, 0)),
        pl.BlockSpec((1, H), lambda i: (0, 0)),
        pl.BlockSpec((H, H), lambda i: (0, 0)),
        pl.BlockSpec((1, H), lambda i: (0, 0)),
    ],
    out_specs=[
        pl.BlockSpec((BN, H), lambda i: (i, 0)),
        pl.BlockSpec((2, H), lambda i: (0, 0)),
    ],
    out_shape=[
        jax.ShapeDtypeStruct((N, H), jnp.float32),
        jax.ShapeDtypeStruct((2, H), jnp.float32),
    ],
)


def _bn_relu_body(z_ref, st_ref, g_ref, be_ref, h_ref):
    mean = st_ref[0:1] * (1.0 / N)
    var = st_ref[1:2] * (1.0 / N) - mean * mean
    inv = lax.rsqrt(var + 1e-5)
    h_ref[...] = jnp.maximum(
        (z_ref[...] - mean) * (inv * g_ref[...]) + be_ref[...], 0.0)


_bn_relu = pl.pallas_call(
    _bn_relu_body,
    grid=(NB,),
    in_specs=[
        pl.BlockSpec((BN, H), lambda i: (i, 0)),
        pl.BlockSpec((2, H), lambda i: (0, 0)),
        pl.BlockSpec((1, H), lambda i: (0, 0)),
        pl.BlockSpec((1, H), lambda i: (0, 0)),
    ],
    out_specs=pl.BlockSpec((BN, H), lambda i: (i, 0)),
    out_shape=jax.ShapeDtypeStruct((N, H), jnp.float32),
)


def _pool_head_body(z_ref, st_ref, g_ref, be_ref, b_ref, hw1_ref, hb1_ref,
                    hw2_ref, hb2_ref, o_ref, acc_ref):
    i = pl.program_id(0)
    mean = st_ref[0:1] * (1.0 / N)
    var = st_ref[1:2] * (1.0 / N) - mean * mean
    inv = lax.rsqrt(var + 1e-5)
    h = jnp.maximum((z_ref[...] - mean) * (inv * g_ref[...]) + be_ref[...],
                    0.0)
    bvals = jnp.broadcast_to(b_ref[0], (G, BN))
    oh = (bvals == lax.broadcasted_iota(jnp.int32, (G, BN), 0)
          ).astype(jnp.float32)
    part = jnp.dot(oh, h, preferred_element_type=jnp.float32)

    @pl.when(i == 0)
    def _():
        acc_ref[...] = part

    @pl.when(i > 0)
    def _():
        acc_ref[...] = acc_ref[...] + part

    @pl.when(i == pl.num_programs(0) - 1)
    def _():
        gpool = acc_ref[...]
        a = jnp.dot(gpool, hw1_ref[...], preferred_element_type=jnp.float32)
        a = jnp.maximum(a + hb1_ref[...], 0.0)
        o = jnp.dot(a, hw2_ref[...], preferred_element_type=jnp.float32)
        o_ref[...] = jax.nn.sigmoid(o + hb2_ref[...])


_pool_head = pl.pallas_call(
    _pool_head_body,
    grid=(NB,),
    in_specs=[
        pl.BlockSpec((BN, H), lambda i: (i, 0)),
        pl.BlockSpec((2, H), lambda i: (0, 0)),
        pl.BlockSpec((1, H), lambda i: (0, 0)),
        pl.BlockSpec((1, H), lambda i: (0, 0)),
        pl.BlockSpec((1, 1, BN), lambda i: (i, 0, 0)),
        pl.BlockSpec((H, H), lambda i: (0, 0)),
        pl.BlockSpec((1, H), lambda i: (0, 0)),
        pl.BlockSpec((H, H), lambda i: (0, 0)),
        pl.BlockSpec((1, H), lambda i: (0, 0)),
    ],
    out_specs=pl.BlockSpec((G, H), lambda i: (0, 0)),
    out_shape=jax.ShapeDtypeStruct((G, H), jnp.float32),
    scratch_shapes=[pltpu.VMEM((G, H), jnp.float32)],
)


def kernel(x, params, edge_index, batch):
    src = edge_index[0]
    dst = edge_index[1]
    zeros = jnp.zeros((N, D), jnp.float32)
    batch3 = batch.reshape(NB, 1, BN)

    hw1p = jnp.zeros((H, H), jnp.float32).at[:, :H // 2].set(params["hw1"])
    hb1p = jnp.zeros((1, H), jnp.float32).at[0, :H // 2].set(params["hb1"])
    hw2p = jnp.zeros((H, H), jnp.float32).at[:H // 2, :OUT].set(params["hw2"])
    hb2p = jnp.zeros((1, H), jnp.float32).at[0, :OUT].set(params["hb2"])

    h = x
    z = st = None
    for i in range(NUM_LAYERS):
        a0, a1 = _sc_agg(h, src, dst, zeros)
        z, st = _gin_mlp(h, a0, a1,
                         params[f"w1_{i}"], params[f"b1_{i}"].reshape(1, H),
                         params[f"w2_{i}"], params[f"b2_{i}"].reshape(1, H))
        if i < NUM_LAYERS - 1:
            h = _bn_relu(z, st, params[f"gamma_{i}"].reshape(1, H),
                         params[f"beta_{i}"].reshape(1, H))

    out = _pool_head(z, st,
                     params[f"gamma_{NUM_LAYERS - 1}"].reshape(1, H),
                     params[f"beta_{NUM_LAYERS - 1}"].reshape(1, H),
                     batch3, hw1p, hb1p, hw2p, hb2p)
    return out[:, :OUT]


# speedup vs baseline: 8.2953x; 1.0238x over previous
import functools

import jax
import jax.numpy as jnp
from jax import lax
from jax.experimental import pallas as pl
from jax.experimental.pallas import tpu as pltpu
from jax.experimental.pallas import tpu_sc as plsc

N = 10000
E = 320000
D = 128
H = 128
G = 64
OUT = 12
NUM_LAYERS = 3

NC = 2
NS = 16
EDGES_PER_CORE = E // NC
EDGES_PER_TILE = EDGES_PER_CORE // NS
CH = 96
NCHUNK = EDGES_PER_TILE // CH
TAIL = EDGES_PER_TILE - NCHUNK * CH
ROWS_PER_TILE = 632

BN = 1000
NB = N // BN


_sc_mesh = plsc.VectorSubcoreMesh(core_axis_name="c", subcore_axis_name="s")


@functools.partial(
    pl.kernel,
    mesh=_sc_mesh,
    out_type=(
        jax.ShapeDtypeStruct((N, D), jnp.float32),
        jax.ShapeDtypeStruct((N, D), jnp.float32),
    ),
    scratch_types=[
        pltpu.VMEM((EDGES_PER_TILE,), jnp.int32),
        pltpu.VMEM((EDGES_PER_TILE,), jnp.int32),
        pltpu.VMEM((CH,), jnp.int32),
        pltpu.VMEM((CH,), jnp.int32),
        pltpu.VMEM((CH, D), jnp.float32),
        pltpu.VMEM((CH, D), jnp.float32),
        pltpu.VMEM_SHARED((N, D), jnp.float32),
        pltpu.SemaphoreType.DMA,
        pltpu.SemaphoreType.DMA,
        pltpu.SemaphoreType.DMA,
        pltpu.SemaphoreType.DMA,
    ],
)
def _sc_agg(h_hbm, src_hbm, dst_hbm, zeros_hbm, out0, out1, srcf_v, dstf_v,
            di0_v, di1_v, rows0_v, rows1_v, acc_sh, sg0, sg1, ss0, ss1):
    c = lax.axis_index("c")
    s = lax.axis_index("s")

    base = c * EDGES_PER_CORE + s * EDGES_PER_TILE
    pltpu.sync_copy(src_hbm.at[pl.ds(base, EDGES_PER_TILE)], srcf_v)
    pltpu.sync_copy(dst_hbm.at[pl.ds(base, EDGES_PER_TILE)], dstf_v)

    r0 = jnp.minimum(s * ROWS_PER_TILE, N - ROWS_PER_TILE)
    pltpu.sync_copy(zeros_hbm.at[pl.ds(r0, ROWS_PER_TILE)],
                    acc_sh.at[pl.ds(r0, ROWS_PER_TILE)])
    plsc.subcore_barrier()

    def chunk_pair(k, _):
        o0 = 2 * k * CH
        o1 = o0 + CH
        g0 = pltpu.async_copy(h_hbm.at[srcf_v.at[pl.ds(o0, CH)]], rows0_v,
                              sg0)
        g1 = pltpu.async_copy(h_hbm.at[srcf_v.at[pl.ds(o1, CH)]], rows1_v,
                              sg1)
        g0.wait()
        s0 = pltpu.async_copy(rows0_v, acc_sh.at[dstf_v.at[pl.ds(o0, CH)]],
                              ss0, add=True)
        g1.wait()
        s1 = pltpu.async_copy(rows1_v, acc_sh.at[dstf_v.at[pl.ds(o1, CH)]],
                              ss1, add=True)
        s0.wait()
        s1.wait()
        return 0

    lax.fori_loop(0, NCHUNK // 2, chunk_pair, 0)
    ot = NCHUNK * CH
    pltpu.async_copy(h_hbm.at[srcf_v.at[pl.ds(ot, TAIL)]],
                     rows0_v.at[pl.ds(0, TAIL)], sg0).wait()
    pltpu.sync_copy(rows0_v.at[pl.ds(0, TAIL)],
                    acc_sh.at[dstf_v.at[pl.ds(ot, TAIL)]], add=True)
    plsc.subcore_barrier()

    @pl.when(c == 0)
    def _():
        pltpu.sync_copy(acc_sh.at[pl.ds(r0, ROWS_PER_TILE)],
                        out0.at[pl.ds(r0, ROWS_PER_TILE)])

    @pl.when(c == 1)
    def _():
        pltpu.sync_copy(acc_sh.at[pl.ds(r0, ROWS_PER_TILE)],
                        out1.at[pl.ds(r0, ROWS_PER_TILE)])


def _gin_mlp_body(h_ref, a0_ref, a1_ref, w1_ref, b1_ref, w2_ref, b2_ref,
                  z_ref, st_ref):
    i = pl.program_id(0)
    h2 = h_ref[...] + a0_ref[...] + a1_ref[...]
    t = jnp.dot(h2, w1_ref[...], preferred_element_type=jnp.float32)
    t = jnp.maximum(t + b1_ref[...], 0.0)
    z = jnp.dot(t, w2_ref[...], preferred_element_type=jnp.float32)
    z = z + b2_ref[...]
    z_ref[...] = z
    st = jnp.concatenate(
        [jnp.sum(z, axis=0, keepdims=True),
         jnp.sum(z * z, axis=0, keepdims=True)], axis=0)

    @pl.when(i == 0)
    def _():
        st_ref[...] = st

    @pl.when(i > 0)
    def _():
        st_ref[...] = st_ref[...] + st


_gin_mlp = pl.pallas_call(
    _gin_mlp_body,
    grid=(NB,),
    in_specs=[
        pl.BlockSpec((BN, D), lambda i: (i, 0)),
        pl.BlockSpec((BN, D), lambda i: (i, 0)),
        pl.BlockSpec((BN, D), lambda i: (i, 0)),
        pl.BlockSpec((D, H), lambda i: (0, 0)),
        pl.BlockSpec((1, H), lambda i: (0, 0)),
        pl.BlockSpec((H, H), lambda i: (0, 0)),
        pl.BlockSpec((1, H), lambda i: (0, 0)),
    ],
    out_specs=[
        pl.BlockSpec((BN, H), lambda i: (i, 0)),
        pl.BlockSpec((2, H), lambda i: (0, 0)),
    ],
    out_shape=[
        jax.ShapeDtypeStruct((N, H), jnp.float32),
        jax.ShapeDtypeStruct((2, H), jnp.float32),
    ],
)


def _bn_relu_body(z_ref, st_ref, g_ref, be_ref, h_ref):
    mean = st_ref[0:1] * (1.0 / N)
    var = st_ref[1:2] * (1.0 / N) - mean * mean
    inv = lax.rsqrt(var + 1e-5)
    h_ref[...] = jnp.maximum(
        (z_ref[...] - mean) * (inv * g_ref[...]) + be_ref[...], 0.0)


_bn_relu = pl.pallas_call(
    _bn_relu_body,
    grid=(NB,),
    in_specs=[
        pl.BlockSpec((BN, H), lambda i: (i, 0)),
        pl.BlockSpec((2, H), lambda i: (0, 0)),
        pl.BlockSpec((1, H), lambda i: (0, 0)),
        pl.BlockSpec((1, H), lambda i: (0, 0)),
    ],
    out_specs=pl.BlockSpec((BN, H), lambda i: (i, 0)),
    out_shape=jax.ShapeDtypeStruct((N, H), jnp.float32),
)


def _pool_head_body(z_ref, st_ref, g_ref, be_ref, b_ref, hw1_ref, hb1_ref,
                    hw2_ref, hb2_ref, o_ref, acc_ref):
    i = pl.program_id(0)
    mean = st_ref[0:1] * (1.0 / N)
    var = st_ref[1:2] * (1.0 / N) - mean * mean
    inv = lax.rsqrt(var + 1e-5)
    h = jnp.maximum((z_ref[...] - mean) * (inv * g_ref[...]) + be_ref[...],
                    0.0)
    bvals = jnp.broadcast_to(b_ref[0], (G, BN))
    oh = (bvals == lax.broadcasted_iota(jnp.int32, (G, BN), 0)
          ).astype(jnp.float32)
    part = jnp.dot(oh, h, preferred_element_type=jnp.float32)

    @pl.when(i == 0)
    def _():
        acc_ref[...] = part

    @pl.when(i > 0)
    def _():
        acc_ref[...] = acc_ref[...] + part

    @pl.when(i == pl.num_programs(0) - 1)
    def _():
        gpool = acc_ref[...]
        a = jnp.dot(gpool, hw1_ref[...], preferred_element_type=jnp.float32)
        a = jnp.maximum(a + hb1_ref[...], 0.0)
        o = jnp.dot(a, hw2_ref[...], preferred_element_type=jnp.float32)
        o_ref[...] = jax.nn.sigmoid(o + hb2_ref[...])


_pool_head = pl.pallas_call(
    _pool_head_body,
    grid=(NB,),
    in_specs=[
        pl.BlockSpec((BN, H), lambda i: (i, 0)),
        pl.BlockSpec((2, H), lambda i: (0, 0)),
        pl.BlockSpec((1, H), lambda i: (0, 0)),
        pl.BlockSpec((1, H), lambda i: (0, 0)),
        pl.BlockSpec((1, 1, BN), lambda i: (i, 0, 0)),
        pl.BlockSpec((H, H), lambda i: (0, 0)),
        pl.BlockSpec((1, H), lambda i: (0, 0)),
        pl.BlockSpec((H, H), lambda i: (0, 0)),
        pl.BlockSpec((1, H), lambda i: (0, 0)),
    ],
    out_specs=pl.BlockSpec((G, H), lambda i: (0, 0)),
    out_shape=jax.ShapeDtypeStruct((G, H), jnp.float32),
    scratch_shapes=[pltpu.VMEM((G, H), jnp.float32)],
)


def kernel(x, params, edge_index, batch):
    src = edge_index[0]
    dst = edge_index[1]
    zeros = jnp.zeros((N, D), jnp.float32)
    batch3 = batch.reshape(NB, 1, BN)

    hw1p = jnp.zeros((H, H), jnp.float32).at[:, :H // 2].set(params["hw1"])
    hb1p = jnp.zeros((1, H), jnp.float32).at[0, :H // 2].set(params["hb1"])
    hw2p = jnp.zeros((H, H), jnp.float32).at[:H // 2, :OUT].set(params["hw2"])
    hb2p = jnp.zeros((1, H), jnp.float32).at[0, :OUT].set(params["hb2"])

    h = x
    z = st = None
    for i in range(NUM_LAYERS):
        a0, a1 = _sc_agg(h, src, dst, zeros)
        z, st = _gin_mlp(h, a0, a1,
                         params[f"w1_{i}"], params[f"b1_{i}"].reshape(1, H),
                         params[f"w2_{i}"], params[f"b2_{i}"].reshape(1, H))
        if i < NUM_LAYERS - 1:
            h = _bn_relu(z, st, params[f"gamma_{i}"].reshape(1, H),
                         params[f"beta_{i}"].reshape(1, H))

    out = _pool_head(z, st,
                     params[f"gamma_{NUM_LAYERS - 1}"].reshape(1, H),
                     params[f"beta_{NUM_LAYERS - 1}"].reshape(1, H),
                     batch3, hw1p, hb1p, hw2p, hb2p)
    return out[:, :OUT]


# speedup vs baseline: 8.5597x; 1.0319x over previous
import functools

import jax
import jax.numpy as jnp
from jax import lax
from jax.experimental import pallas as pl
from jax.experimental.pallas import tpu as pltpu
from jax.experimental.pallas import tpu_sc as plsc

N = 10000
E = 320000
D = 128
H = 128
G = 64
OUT = 12
NUM_LAYERS = 3

NC = 2
NS = 16
EDGES_PER_CORE = E // NC
EDGES_PER_TILE = EDGES_PER_CORE // NS
CH = 128
NCHUNK = EDGES_PER_TILE // CH
TAIL = EDGES_PER_TILE - NCHUNK * CH
ROWS_PER_TILE = 632

BN = 1000
NB = N // BN


_sc_mesh = plsc.VectorSubcoreMesh(core_axis_name="c", subcore_axis_name="s")


NGROUP = NCHUNK // 2
NITER = NGROUP // 2
QSPAN = 2 * CH


@functools.partial(
    pl.kernel,
    mesh=_sc_mesh,
    out_type=(
        jax.ShapeDtypeStruct((N, D), jnp.float32),
        jax.ShapeDtypeStruct((N, D), jnp.float32),
    ),
    scratch_types=[
        [pltpu.VMEM((CH, D), jnp.float32) for _ in range(2)],
        [[pltpu.VMEM((CH,), jnp.int32) for _ in range(2)] for _ in range(2)],
        [[pltpu.VMEM((CH,), jnp.int32) for _ in range(2)] for _ in range(2)],
        pltpu.VMEM((TAIL,), jnp.int32),
        pltpu.VMEM((TAIL,), jnp.int32),
        pltpu.VMEM_SHARED((N, D), jnp.float32),
        [pltpu.SemaphoreType.DMA for _ in range(2)],
        [pltpu.SemaphoreType.DMA for _ in range(2)],
        [[pltpu.SemaphoreType.DMA for _ in range(2)] for _ in range(2)],
    ],
)
def _sc_agg(h_hbm, src_hbm, dst_hbm, zeros_hbm, out0, out1, rows, si, di,
            st_v, dt_v, acc_sh, sg, ss, sidx):
    c = lax.axis_index("c")
    s = lax.axis_index("s")

    base = c * EDGES_PER_CORE + s * EDGES_PER_TILE

    def load_idx(bank, gbase):
        for b in range(2):
            off = gbase + b * CH
            pltpu.async_copy(src_hbm.at[pl.ds(off, CH)], si[bank][b],
                             sidx[bank][b])
            pltpu.async_copy(dst_hbm.at[pl.ds(off, CH)], di[bank][b],
                             sidx[bank][b])

    def wait_idx(bank, gbase):
        for b in range(2):
            off = gbase + b * CH
            pltpu.make_async_copy(src_hbm.at[pl.ds(off, CH)], si[bank][b],
                                  sidx[bank][b]).wait()
            pltpu.make_async_copy(dst_hbm.at[pl.ds(off, CH)], di[bank][b],
                                  sidx[bank][b]).wait()

    def do_group(bank, gbase):
        wait_idx(bank, gbase)
        return [pltpu.async_copy(h_hbm.at[si[bank][b]], rows[b], sg[b])
                for b in range(2)]

    def drain_group(bank, gs):
        descs = []
        for b in range(2):
            gs[b].wait()
            descs.append(pltpu.async_copy(rows[b], acc_sh.at[di[bank][b]],
                                          ss[b], add=True))
        for d in descs:
            d.wait()

    load_idx(0, base)

    r0 = jnp.minimum(s * ROWS_PER_TILE, N - ROWS_PER_TILE)
    pltpu.sync_copy(zeros_hbm.at[pl.ds(r0, ROWS_PER_TILE)],
                    acc_sh.at[pl.ds(r0, ROWS_PER_TILE)])
    plsc.subcore_barrier()

    def group_pair(k, _):
        gb0 = base + k * (2 * QSPAN)
        gb1 = gb0 + QSPAN
        gs = do_group(0, gb0)
        load_idx(1, gb1)
        drain_group(0, gs)
        gs = do_group(1, gb1)
        load_idx(0, gb1 + QSPAN)
        drain_group(1, gs)
        return 0

    lax.fori_loop(0, NITER, group_pair, 0)

    gbl = base + (NGROUP - 1) * QSPAN
    gs = do_group(0, gbl)
    drain_group(0, gs)

    ot = base + NCHUNK * CH
    pltpu.sync_copy(src_hbm.at[pl.ds(ot, TAIL)], st_v)
    pltpu.sync_copy(dst_hbm.at[pl.ds(ot, TAIL)], dt_v)
    pltpu.async_copy(h_hbm.at[st_v], rows[0].at[pl.ds(0, TAIL)], sg[0]).wait()
    pltpu.sync_copy(rows[0].at[pl.ds(0, TAIL)], acc_sh.at[dt_v], add=True)
    plsc.subcore_barrier()

    @pl.when(c == 0)
    def _():
        pltpu.sync_copy(acc_sh.at[pl.ds(r0, ROWS_PER_TILE)],
                        out0.at[pl.ds(r0, ROWS_PER_TILE)])

    @pl.when(c == 1)
    def _():
        pltpu.sync_copy(acc_sh.at[pl.ds(r0, ROWS_PER_TILE)],
                        out1.at[pl.ds(r0, ROWS_PER_TILE)])


def _gin_mlp_body(h_ref, a0_ref, a1_ref, w1_ref, b1_ref, w2_ref, b2_ref,
                  z_ref, st_ref):
    i = pl.program_id(0)
    h2 = h_ref[...] + a0_ref[...] + a1_ref[...]
    t = jnp.dot(h2, w1_ref[...], preferred_element_type=jnp.float32)
    t = jnp.maximum(t + b1_ref[...], 0.0)
    z = jnp.dot(t, w2_ref[...], preferred_element_type=jnp.float32)
    z = z + b2_ref[...]
    z_ref[...] = z
    st = jnp.concatenate(
        [jnp.sum(z, axis=0, keepdims=True),
         jnp.sum(z * z, axis=0, keepdims=True)], axis=0)

    @pl.when(i == 0)
    def _():
        st_ref[...] = st

    @pl.when(i > 0)
    def _():
        st_ref[...] = st_ref[...] + st


_gin_mlp = pl.pallas_call(
    _gin_mlp_body,
    grid=(NB,),
    in_specs=[
        pl.BlockSpec((BN, D), lambda i: (i, 0)),
        pl.BlockSpec((BN, D), lambda i: (i, 0)),
        pl.BlockSpec((BN, D), lambda i: (i, 0)),
        pl.BlockSpec((D, H), lambda i: (0, 0)),
        pl.BlockSpec((1, H), lambda i: (0, 0)),
        pl.BlockSpec((H, H), lambda i: (0, 0)),
        pl.BlockSpec((1, H), lambda i: (0, 0)),
    ],
    out_specs=[
        pl.BlockSpec((BN, H), lambda i: (i, 0)),
        pl.BlockSpec((2, H), lambda i: (0, 0)),
    ],
    out_shape=[
        jax.ShapeDtypeStruct((N, H), jnp.float32),
        jax.ShapeDtypeStruct((2, H), jnp.float32),
    ],
)


def _bn_relu_body(z_ref, st_ref, g_ref, be_ref, h_ref):
    mean = st_ref[0:1] * (1.0 / N)
    var = st_ref[1:2] * (1.0 / N) - mean * mean
    inv = lax.rsqrt(var + 1e-5)
    h_ref[...] = jnp.maximum(
        (z_ref[...] - mean) * (inv * g_ref[...]) + be_ref[...], 0.0)


_bn_relu = pl.pallas_call(
    _bn_relu_body,
    grid=(NB,),
    in_specs=[
        pl.BlockSpec((BN, H), lambda i: (i, 0)),
        pl.BlockSpec((2, H), lambda i: (0, 0)),
        pl.BlockSpec((1, H), lambda i: (0, 0)),
        pl.BlockSpec((1, H), lambda i: (0, 0)),
    ],
    out_specs=pl.BlockSpec((BN, H), lambda i: (i, 0)),
    out_shape=jax.ShapeDtypeStruct((N, H), jnp.float32),
)


def _pool_head_body(z_ref, st_ref, g_ref, be_ref, b_ref, hw1_ref, hb1_ref,
                    hw2_ref, hb2_ref, o_ref, acc_ref):
    i = pl.program_id(0)
    mean = st_ref[0:1] * (1.0 / N)
    var = st_ref[1:2] * (1.0 / N) - mean * mean
    inv = lax.rsqrt(var + 1e-5)
    h = jnp.maximum((z_ref[...] - mean) * (inv * g_ref[...]) + be_ref[...],
                    0.0)
    bvals = jnp.broadcast_to(b_ref[0], (G, BN))
    oh = (bvals == lax.broadcasted_iota(jnp.int32, (G, BN), 0)
          ).astype(jnp.float32)
    part = jnp.dot(oh, h, preferred_element_type=jnp.float32)

    @pl.when(i == 0)
    def _():
        acc_ref[...] = part

    @pl.when(i > 0)
    def _():
        acc_ref[...] = acc_ref[...] + part

    @pl.when(i == pl.num_programs(0) - 1)
    def _():
        gpool = acc_ref[...]
        a = jnp.dot(gpool, hw1_ref[...], preferred_element_type=jnp.float32)
        a = jnp.maximum(a + hb1_ref[...], 0.0)
        o = jnp.dot(a, hw2_ref[...], preferred_element_type=jnp.float32)
        o_ref[...] = jax.nn.sigmoid(o + hb2_ref[...])


_pool_head = pl.pallas_call(
    _pool_head_body,
    grid=(NB,),
    in_specs=[
        pl.BlockSpec((BN, H), lambda i: (i, 0)),
        pl.BlockSpec((2, H), lambda i: (0, 0)),
        pl.BlockSpec((1, H), lambda i: (0, 0)),
        pl.BlockSpec((1, H), lambda i: (0, 0)),
        pl.BlockSpec((1, 1, BN), lambda i: (i, 0, 0)),
        pl.BlockSpec((H, H), lambda i: (0, 0)),
        pl.BlockSpec((1, H), lambda i: (0, 0)),
        pl.BlockSpec((H, H), lambda i: (0, 0)),
        pl.BlockSpec((1, H), lambda i: (0, 0)),
    ],
    out_specs=pl.BlockSpec((G, H), lambda i: (0, 0)),
    out_shape=jax.ShapeDtypeStruct((G, H), jnp.float32),
    scratch_shapes=[pltpu.VMEM((G, H), jnp.float32)],
)


def kernel(x, params, edge_index, batch):
    src = edge_index[0]
    dst = edge_index[1]
    zeros = jnp.zeros((N, D), jnp.float32)
    batch3 = batch.reshape(NB, 1, BN)

    hw1p = jnp.zeros((H, H), jnp.float32).at[:, :H // 2].set(params["hw1"])
    hb1p = jnp.zeros((1, H), jnp.float32).at[0, :H // 2].set(params["hb1"])
    hw2p = jnp.zeros((H, H), jnp.float32).at[:H // 2, :OUT].set(params["hw2"])
    hb2p = jnp.zeros((1, H), jnp.float32).at[0, :OUT].set(params["hb2"])

    h = x
    z = st = None
    for i in range(NUM_LAYERS):
        a0, a1 = _sc_agg(h, src, dst, zeros)
        z, st = _gin_mlp(h, a0, a1,
                         params[f"w1_{i}"], params[f"b1_{i}"].reshape(1, H),
                         params[f"w2_{i}"], params[f"b2_{i}"].reshape(1, H))
        if i < NUM_LAYERS - 1:
            h = _bn_relu(z, st, params[f"gamma_{i}"].reshape(1, H),
                         params[f"beta_{i}"].reshape(1, H))

    out = _pool_head(z, st,
                     params[f"gamma_{NUM_LAYERS - 1}"].reshape(1, H),
                     params[f"beta_{NUM_LAYERS - 1}"].reshape(1, H),
                     batch3, hw1p, hb1p, hw2p, hb2p)
    return out[:, :OUT]


# speedup vs baseline: 10.0128x; 1.1698x over previous
import functools

import jax
import jax.numpy as jnp
from jax import lax
from jax.experimental import pallas as pl
from jax.experimental.pallas import tpu as pltpu
from jax.experimental.pallas import tpu_sc as plsc

N = 10000
E = 320000
D = 128
H = 128
G = 64
OUT = 12
NUM_LAYERS = 3

NC = 2
NS = 16
EDGES_PER_CORE = E // NC
EDGES_PER_TILE = EDGES_PER_CORE // NS
CH = 104
NCHUNK = 96
TAIL = EDGES_PER_TILE - NCHUNK * CH
ROWS_PER_TILE = 632

BN = 1000
NB = N // BN


_sc_mesh = plsc.VectorSubcoreMesh(core_axis_name="c", subcore_axis_name="s")


NROT = 3
NITER = NCHUNK // (2 * NROT)


@functools.partial(
    pl.kernel,
    mesh=_sc_mesh,
    out_type=(
        jax.ShapeDtypeStruct((N, D), jnp.float32),
        jax.ShapeDtypeStruct((N, D), jnp.float32),
    ),
    scratch_types=[
        [pltpu.VMEM((CH, D), jnp.float32) for _ in range(NROT)],
        [[pltpu.VMEM((CH,), jnp.int32) for _ in range(NROT)] for _ in range(2)],
        [[pltpu.VMEM((CH,), jnp.int32) for _ in range(NROT)] for _ in range(2)],
        pltpu.VMEM((TAIL,), jnp.int32),
        pltpu.VMEM((TAIL,), jnp.int32),
        pltpu.VMEM_SHARED((N, D), jnp.float32),
        [pltpu.SemaphoreType.DMA for _ in range(NROT)],
        [pltpu.SemaphoreType.DMA for _ in range(NROT)],
        [[pltpu.SemaphoreType.DMA for _ in range(NROT)] for _ in range(2)],
    ],
)
def _sc_agg(h_hbm, src_hbm, dst_hbm, zeros_hbm, out0, out1, rows, si, di,
            st_v, dt_v, acc_sh, sg, ss, sidx):
    c = lax.axis_index("c")
    s = lax.axis_index("s")

    base = c * EDGES_PER_CORE + s * EDGES_PER_TILE

    def load_idx(p, b, off):
        pltpu.async_copy(src_hbm.at[pl.ds(off, CH)], si[p][b], sidx[p][b])
        pltpu.async_copy(dst_hbm.at[pl.ds(off, CH)], di[p][b], sidx[p][b])

    def wait_idx(p, b, off):
        pltpu.make_async_copy(src_hbm.at[pl.ds(off, CH)], si[p][b],
                              sidx[p][b]).wait()
        pltpu.make_async_copy(dst_hbm.at[pl.ds(off, CH)], di[p][b],
                              sidx[p][b]).wait()

    def wait_scatter(p, b):
        pltpu.make_async_copy(rows[b], acc_sh.at[di[p][b]], ss[b]).wait()

    for b in range(NROT):
        load_idx(0, b, base + b * CH)

    r0 = jnp.minimum(s * ROWS_PER_TILE, N - ROWS_PER_TILE)
    pltpu.sync_copy(zeros_hbm.at[pl.ds(r0, ROWS_PER_TILE)],
                    acc_sh.at[pl.ds(r0, ROWS_PER_TILE)])
    plsc.subcore_barrier()

    def body(k, _):
        for p in range(2):
            rot = 2 * k + p
            rbase = base + rot * (NROT * CH)
            gs = []
            for b in range(NROT):
                if p == 0:
                    @pl.when(k > 0)
                    def _(b=b):
                        wait_scatter(1, b)
                else:
                    wait_scatter(0, b)
                wait_idx(p, b, rbase + b * CH)
                gs.append(pltpu.async_copy(h_hbm.at[si[p][b]], rows[b],
                                           sg[b]))
            if p == 0:
                for b in range(NROT):
                    load_idx(1, b, rbase + (NROT + b) * CH)
            else:
                @pl.when(k < NITER - 1)
                def _():
                    for b in range(NROT):
                        load_idx(0, b, rbase + (NROT + b) * CH)
            for b in range(NROT):
                gs[b].wait()
                pltpu.async_copy(rows[b], acc_sh.at[di[p][b]], ss[b],
                                 add=True)
        return 0

    lax.fori_loop(0, NITER, body, 0)
    for b in range(NROT):
        wait_scatter(1, b)

    ot = base + NCHUNK * CH
    pltpu.sync_copy(src_hbm.at[pl.ds(ot, TAIL)], st_v)
    pltpu.sync_copy(dst_hbm.at[pl.ds(ot, TAIL)], dt_v)
    pltpu.async_copy(h_hbm.at[st_v], rows[0].at[pl.ds(0, TAIL)], sg[0]).wait()
    pltpu.sync_copy(rows[0].at[pl.ds(0, TAIL)], acc_sh.at[dt_v], add=True)
    plsc.subcore_barrier()

    @pl.when(c == 0)
    def _():
        pltpu.sync_copy(acc_sh.at[pl.ds(r0, ROWS_PER_TILE)],
                        out0.at[pl.ds(r0, ROWS_PER_TILE)])

    @pl.when(c == 1)
    def _():
        pltpu.sync_copy(acc_sh.at[pl.ds(r0, ROWS_PER_TILE)],
                        out1.at[pl.ds(r0, ROWS_PER_TILE)])


def _gin_mlp_body(h_ref, a0_ref, a1_ref, w1_ref, b1_ref, w2_ref, b2_ref,
                  z_ref, st_ref):
    i = pl.program_id(0)
    h2 = h_ref[...] + a0_ref[...] + a1_ref[...]
    t = jnp.dot(h2, w1_ref[...], preferred_element_type=jnp.float32)
    t = jnp.maximum(t + b1_ref[...], 0.0)
    z = jnp.dot(t, w2_ref[...], preferred_element_type=jnp.float32)
    z = z + b2_ref[...]
    z_ref[...] = z
    st = jnp.concatenate(
        [jnp.sum(z, axis=0, keepdims=True),
         jnp.sum(z * z, axis=0, keepdims=True)], axis=0)

    @pl.when(i == 0)
    def _():
        st_ref[...] = st

    @pl.when(i > 0)
    def _():
        st_ref[...] = st_ref[...] + st


_gin_mlp = pl.pallas_call(
    _gin_mlp_body,
    grid=(NB,),
    in_specs=[
        pl.BlockSpec((BN, D), lambda i: (i, 0)),
        pl.BlockSpec((BN, D), lambda i: (i, 0)),
        pl.BlockSpec((BN, D), lambda i: (i, 0)),
        pl.BlockSpec((D, H), lambda i: (0, 0)),
        pl.BlockSpec((1, H), lambda i: (0, 0)),
        pl.BlockSpec((H, H), lambda i: (0, 0)),
        pl.BlockSpec((1, H), lambda i: (0, 0)),
    ],
    out_specs=[
        pl.BlockSpec((BN, H), lambda i: (i, 0)),
        pl.BlockSpec((2, H), lambda i: (0, 0)),
    ],
    out_shape=[
        jax.ShapeDtypeStruct((N, H), jnp.float32),
        jax.ShapeDtypeStruct((2, H), jnp.float32),
    ],
)


def _bn_relu_body(z_ref, st_ref, g_ref, be_ref, h_ref):
    mean = st_ref[0:1] * (1.0 / N)
    var = st_ref[1:2] * (1.0 / N) - mean * mean
    inv = lax.rsqrt(var + 1e-5)
    h_ref[...] = jnp.maximum(
        (z_ref[...] - mean) * (inv * g_ref[...]) + be_ref[...], 0.0)


_bn_relu = pl.pallas_call(
    _bn_relu_body,
    grid=(NB,),
    in_specs=[
        pl.BlockSpec((BN, H), lambda i: (i, 0)),
        pl.BlockSpec((2, H), lambda i: (0, 0)),
        pl.BlockSpec((1, H), lambda i: (0, 0)),
        pl.BlockSpec((1, H), lambda i: (0, 0)),
    ],
    out_specs=pl.BlockSpec((BN, H), lambda i: (i, 0)),
    out_shape=jax.ShapeDtypeStruct((N, H), jnp.float32),
)


def _pool_head_body(z_ref, st_ref, g_ref, be_ref, b_ref, hw1_ref, hb1_ref,
                    hw2_ref, hb2_ref, o_ref, acc_ref):
    i = pl.program_id(0)
    mean = st_ref[0:1] * (1.0 / N)
    var = st_ref[1:2] * (1.0 / N) - mean * mean
    inv = lax.rsqrt(var + 1e-5)
    h = jnp.maximum((z_ref[...] - mean) * (inv * g_ref[...]) + be_ref[...],
                    0.0)
    bvals = jnp.broadcast_to(b_ref[0], (G, BN))
    oh = (bvals == lax.broadcasted_iota(jnp.int32, (G, BN), 0)
          ).astype(jnp.float32)
    part = jnp.dot(oh, h, preferred_element_type=jnp.float32)

    @pl.when(i == 0)
    def _():
        acc_ref[...] = part

    @pl.when(i > 0)
    def _():
        acc_ref[...] = acc_ref[...] + part

    @pl.when(i == pl.num_programs(0) - 1)
    def _():
        gpool = acc_ref[...]
        a = jnp.dot(gpool, hw1_ref[...], preferred_element_type=jnp.float32)
        a = jnp.maximum(a + hb1_ref[...], 0.0)
        o = jnp.dot(a, hw2_ref[...], preferred_element_type=jnp.float32)
        o_ref[...] = jax.nn.sigmoid(o + hb2_ref[...])


_pool_head = pl.pallas_call(
    _pool_head_body,
    grid=(NB,),
    in_specs=[
        pl.BlockSpec((BN, H), lambda i: (i, 0)),
        pl.BlockSpec((2, H), lambda i: (0, 0)),
        pl.BlockSpec((1, H), lambda i: (0, 0)),
        pl.BlockSpec((1, H), lambda i: (0, 0)),
        pl.BlockSpec((1, 1, BN), lambda i: (i, 0, 0)),
        pl.BlockSpec((H, H), lambda i: (0, 0)),
        pl.BlockSpec((1, H), lambda i: (0, 0)),
        pl.BlockSpec((H, H), lambda i: (0, 0)),
        pl.BlockSpec((1, H), lambda i: (0, 0)),
    ],
    out_specs=pl.BlockSpec((G, H), lambda i: (0, 0)),
    out_shape=jax.ShapeDtypeStruct((G, H), jnp.float32),
    scratch_shapes=[pltpu.VMEM((G, H), jnp.float32)],
)


def kernel(x, params, edge_index, batch):
    src = edge_index[0]
    dst = edge_index[1]
    zeros = jnp.zeros((N, D), jnp.float32)
    batch3 = batch.reshape(NB, 1, BN)

    hw1p = jnp.zeros((H, H), jnp.float32).at[:, :H // 2].set(params["hw1"])
    hb1p = jnp.zeros((1, H), jnp.float32).at[0, :H // 2].set(params["hb1"])
    hw2p = jnp.zeros((H, H), jnp.float32).at[:H // 2, :OUT].set(params["hw2"])
    hb2p = jnp.zeros((1, H), jnp.float32).at[0, :OUT].set(params["hb2"])

    h = x
    z = st = None
    for i in range(NUM_LAYERS):
        a0, a1 = _sc_agg(h, src, dst, zeros)
        z, st = _gin_mlp(h, a0, a1,
                         params[f"w1_{i}"], params[f"b1_{i}"].reshape(1, H),
                         params[f"w2_{i}"], params[f"b2_{i}"].reshape(1, H))
        if i < NUM_LAYERS - 1:
            h = _bn_relu(z, st, params[f"gamma_{i}"].reshape(1, H),
                         params[f"beta_{i}"].reshape(1, H))

    out = _pool_head(z, st,
                     params[f"gamma_{NUM_LAYERS - 1}"].reshape(1, H),
                     params[f"beta_{NUM_LAYERS - 1}"].reshape(1, H),
                     batch3, hw1p, hb1p, hw2p, hb2p)
    return out[:, :OUT]


# speedup vs baseline: 10.2642x; 1.0251x over previous
import functools

import jax
import jax.numpy as jnp
from jax import lax
from jax.experimental import pallas as pl
from jax.experimental.pallas import tpu as pltpu
from jax.experimental.pallas import tpu_sc as plsc

N = 10000
E = 320000
D = 128
H = 128
G = 64
OUT = 12
NUM_LAYERS = 3

NC = 2
NS = 16
EDGES_PER_CORE = E // NC
EDGES_PER_TILE = EDGES_PER_CORE // NS
CH = 104
NCHUNK = 96
TAIL = EDGES_PER_TILE - NCHUNK * CH
ROWS_PER_TILE = 632

BN = 1000
NB = N // BN


_sc_mesh = plsc.VectorSubcoreMesh(core_axis_name="c", subcore_axis_name="s")


NROT = 3
NITER = NCHUNK // (2 * NROT)


@functools.partial(
    pl.kernel,
    mesh=_sc_mesh,
    out_type=(
        jax.ShapeDtypeStruct((N, D), jnp.float32),
        jax.ShapeDtypeStruct((N, D), jnp.float32),
    ),
    scratch_types=[
        [pltpu.VMEM((CH, D), jnp.float32) for _ in range(NROT)],
        [[pltpu.VMEM((CH,), jnp.int32) for _ in range(NROT)] for _ in range(2)],
        [[pltpu.VMEM((CH,), jnp.int32) for _ in range(NROT)] for _ in range(2)],
        pltpu.VMEM((TAIL,), jnp.int32),
        pltpu.VMEM((TAIL,), jnp.int32),
        pltpu.VMEM_SHARED((N, D), jnp.float32),
        [pltpu.SemaphoreType.DMA for _ in range(NROT)],
        [pltpu.SemaphoreType.DMA for _ in range(NROT)],
        [[pltpu.SemaphoreType.DMA for _ in range(NROT)] for _ in range(2)],
    ],
)
def _sc_agg(h_hbm, src_hbm, dst_hbm, zeros_hbm, out0, out1, rows, si, di,
            st_v, dt_v, acc_sh, sg, ss, sidx):
    c = lax.axis_index("c")
    s = lax.axis_index("s")

    base = c * EDGES_PER_CORE + s * EDGES_PER_TILE

    def load_idx(p, b, off):
        pltpu.async_copy(src_hbm.at[pl.ds(off, CH)], si[p][b], sidx[p][b])
        pltpu.async_copy(dst_hbm.at[pl.ds(off, CH)], di[p][b], sidx[p][b])

    def wait_idx(p, b, off):
        pltpu.make_async_copy(src_hbm.at[pl.ds(off, CH)], si[p][b],
                              sidx[p][b]).wait()
        pltpu.make_async_copy(dst_hbm.at[pl.ds(off, CH)], di[p][b],
                              sidx[p][b]).wait()

    def wait_scatter(p, b):
        pltpu.make_async_copy(rows[b], acc_sh.at[di[p][b]], ss[b]).wait()

    for b in range(NROT):
        load_idx(0, b, base + b * CH)

    r0 = jnp.minimum(s * ROWS_PER_TILE, N - ROWS_PER_TILE)
    pltpu.sync_copy(zeros_hbm.at[pl.ds(r0, ROWS_PER_TILE)],
                    acc_sh.at[pl.ds(r0, ROWS_PER_TILE)])
    plsc.subcore_barrier()

    def body(k, _):
        for p in range(2):
            rot = 2 * k + p
            rbase = base + rot * (NROT * CH)
            gs = []
            for b in range(NROT):
                if p == 0:
                    @pl.when(k > 0)
                    def _(b=b):
                        wait_scatter(1, b)
                else:
                    wait_scatter(0, b)
                wait_idx(p, b, rbase + b * CH)
                gs.append(pltpu.async_copy(h_hbm.at[si[p][b]], rows[b],
                                           sg[b]))
            if p == 0:
                for b in range(NROT):
                    load_idx(1, b, rbase + (NROT + b) * CH)
            else:
                @pl.when(k < NITER - 1)
                def _():
                    for b in range(NROT):
                        load_idx(0, b, rbase + (NROT + b) * CH)
            for b in range(NROT):
                gs[b].wait()
                pltpu.async_copy(rows[b], acc_sh.at[di[p][b]], ss[b],
                                 add=True)
        return 0

    lax.fori_loop(0, NITER, body, 0)
    for b in range(NROT):
        wait_scatter(1, b)

    ot = base + NCHUNK * CH
    pltpu.sync_copy(src_hbm.at[pl.ds(ot, TAIL)], st_v)
    pltpu.sync_copy(dst_hbm.at[pl.ds(ot, TAIL)], dt_v)
    pltpu.async_copy(h_hbm.at[st_v], rows[0].at[pl.ds(0, TAIL)], sg[0]).wait()
    pltpu.sync_copy(rows[0].at[pl.ds(0, TAIL)], acc_sh.at[dt_v], add=True)
    plsc.subcore_barrier()

    @pl.when(c == 0)
    def _():
        pltpu.sync_copy(acc_sh.at[pl.ds(r0, ROWS_PER_TILE)],
                        out0.at[pl.ds(r0, ROWS_PER_TILE)])

    @pl.when(c == 1)
    def _():
        pltpu.sync_copy(acc_sh.at[pl.ds(r0, ROWS_PER_TILE)],
                        out1.at[pl.ds(r0, ROWS_PER_TILE)])


def _gin_layer_body(h_ref, a0_ref, a1_ref, w1_ref, b1_ref, w2_ref, b2_ref,
                    g_ref, be_ref, h_out_ref, z_ref, st_ref):
    ph = pl.program_id(0)
    j = pl.program_id(1)

    @pl.when(ph == 0)
    def _():
        h2 = h_ref[...] + a0_ref[...] + a1_ref[...]
        t = jnp.dot(h2, w1_ref[...], preferred_element_type=jnp.float32)
        t = jnp.maximum(t + b1_ref[...], 0.0)
        z = jnp.dot(t, w2_ref[...], preferred_element_type=jnp.float32)
        z = z + b2_ref[...]
        z_ref[pl.ds(j * BN, BN), :] = z
        st = jnp.concatenate(
            [jnp.sum(z, axis=0, keepdims=True),
             jnp.sum(z * z, axis=0, keepdims=True)], axis=0)

        @pl.when(j == 0)
        def _():
            st_ref[...] = st

        @pl.when(j > 0)
        def _():
            st_ref[...] = st_ref[...] + st

    @pl.when(ph == 1)
    def _():
        mean = st_ref[0:1] * (1.0 / N)
        var = st_ref[1:2] * (1.0 / N) - mean * mean
        inv = lax.rsqrt(var + 1e-5)
        z = z_ref[pl.ds(j * BN, BN), :]
        h_out_ref[...] = jnp.maximum(
            (z - mean) * (inv * g_ref[...]) + be_ref[...], 0.0)


_gin_layer = pl.pallas_call(
    _gin_layer_body,
    grid=(2, NB),
    in_specs=[
        pl.BlockSpec((BN, D), lambda ph, j: (j * (1 - ph), 0)),
        pl.BlockSpec((BN, D), lambda ph, j: (j * (1 - ph), 0)),
        pl.BlockSpec((BN, D), lambda ph, j: (j * (1 - ph), 0)),
        pl.BlockSpec((D, H), lambda ph, j: (0, 0)),
        pl.BlockSpec((1, H), lambda ph, j: (0, 0)),
        pl.BlockSpec((H, H), lambda ph, j: (0, 0)),
        pl.BlockSpec((1, H), lambda ph, j: (0, 0)),
        pl.BlockSpec((1, H), lambda ph, j: (0, 0)),
        pl.BlockSpec((1, H), lambda ph, j: (0, 0)),
    ],
    out_specs=pl.BlockSpec((BN, H), lambda ph, j: (j, 0)),
    out_shape=jax.ShapeDtypeStruct((N, H), jnp.float32),
    scratch_shapes=[
        pltpu.VMEM((N, H), jnp.float32),
        pltpu.VMEM((2, H), jnp.float32),
    ],
)


def _final_layer_body(h_ref, a0_ref, a1_ref, w1_ref, b1_ref, w2_ref, b2_ref,
                      g_ref, be_ref, b_ref, hw1_ref, hb1_ref, hw2_ref,
                      hb2_ref, o_ref, z_ref, st_ref, acc_ref):
    ph = pl.program_id(0)
    j = pl.program_id(1)

    @pl.when(ph == 0)
    def _():
        h2 = h_ref[...] + a0_ref[...] + a1_ref[...]
        t = jnp.dot(h2, w1_ref[...], preferred_element_type=jnp.float32)
        t = jnp.maximum(t + b1_ref[...], 0.0)
        z = jnp.dot(t, w2_ref[...], preferred_element_type=jnp.float32)
        z = z + b2_ref[...]
        z_ref[pl.ds(j * BN, BN), :] = z
        st = jnp.concatenate(
            [jnp.sum(z, axis=0, keepdims=True),
             jnp.sum(z * z, axis=0, keepdims=True)], axis=0)

        @pl.when(j == 0)
        def _():
            st_ref[...] = st

        @pl.when(j > 0)
        def _():
            st_ref[...] = st_ref[...] + st

    @pl.when(ph == 1)
    def _():
        mean = st_ref[0:1] * (1.0 / N)
        var = st_ref[1:2] * (1.0 / N) - mean * mean
        inv = lax.rsqrt(var + 1e-5)
        z = z_ref[pl.ds(j * BN, BN), :]
        h = jnp.maximum((z - mean) * (inv * g_ref[...]) + be_ref[...], 0.0)
        bvals = jnp.broadcast_to(b_ref[0], (G, BN))
        oh = (bvals == lax.broadcasted_iota(jnp.int32, (G, BN), 0)
              ).astype(jnp.float32)
        part = jnp.dot(oh, h, preferred_element_type=jnp.float32)

        @pl.when(j == 0)
        def _():
            acc_ref[...] = part

        @pl.when(j > 0)
        def _():
            acc_ref[...] = acc_ref[...] + part

        @pl.when(j == NB - 1)
        def _():
            gpool = acc_ref[...]
            a = jnp.dot(gpool, hw1_ref[...],
                        preferred_element_type=jnp.float32)
            a = jnp.maximum(a + hb1_ref[...], 0.0)
            o = jnp.dot(a, hw2_ref[...], preferred_element_type=jnp.float32)
            o_ref[...] = jax.nn.sigmoid(o + hb2_ref[...])


_final_layer = pl.pallas_call(
    _final_layer_body,
    grid=(2, NB),
    in_specs=[
        pl.BlockSpec((BN, D), lambda ph, j: (j * (1 - ph), 0)),
        pl.BlockSpec((BN, D), lambda ph, j: (j * (1 - ph), 0)),
        pl.BlockSpec((BN, D), lambda ph, j: (j * (1 - ph), 0)),
        pl.BlockSpec((D, H), lambda ph, j: (0, 0)),
        pl.BlockSpec((1, H), lambda ph, j: (0, 0)),
        pl.BlockSpec((H, H), lambda ph, j: (0, 0)),
        pl.BlockSpec((1, H), lambda ph, j: (0, 0)),
        pl.BlockSpec((1, H), lambda ph, j: (0, 0)),
        pl.BlockSpec((1, H), lambda ph, j: (0, 0)),
        pl.BlockSpec((1, 1, BN), lambda ph, j: (j * ph, 0, 0)),
        pl.BlockSpec((H, H), lambda ph, j: (0, 0)),
        pl.BlockSpec((1, H), lambda ph, j: (0, 0)),
        pl.BlockSpec((H, H), lambda ph, j: (0, 0)),
        pl.BlockSpec((1, H), lambda ph, j: (0, 0)),
    ],
    out_specs=pl.BlockSpec((G, H), lambda ph, j: (0, 0)),
    out_shape=jax.ShapeDtypeStruct((G, H), jnp.float32),
    scratch_shapes=[
        pltpu.VMEM((N, H), jnp.float32),
        pltpu.VMEM((2, H), jnp.float32),
        pltpu.VMEM((G, H), jnp.float32),
    ],
)


def kernel(x, params, edge_index, batch):
    src = edge_index[0]
    dst = edge_index[1]
    zeros = jnp.zeros((N, D), jnp.float32)
    batch3 = batch.reshape(NB, 1, BN)

    hw1p = jnp.zeros((H, H), jnp.float32).at[:, :H // 2].set(params["hw1"])
    hb1p = jnp.zeros((1, H), jnp.float32).at[0, :H // 2].set(params["hb1"])
    hw2p = jnp.zeros((H, H), jnp.float32).at[:H // 2, :OUT].set(params["hw2"])
    hb2p = jnp.zeros((1, H), jnp.float32).at[0, :OUT].set(params["hb2"])

    h = x
    for i in range(NUM_LAYERS - 1):
        a0, a1 = _sc_agg(h, src, dst, zeros)
        h = _gin_layer(h, a0, a1,
                       params[f"w1_{i}"], params[f"b1_{i}"].reshape(1, H),
                       params[f"w2_{i}"], params[f"b2_{i}"].reshape(1, H),
                       params[f"gamma_{i}"].reshape(1, H),
                       params[f"beta_{i}"].reshape(1, H))

    i = NUM_LAYERS - 1
    a0, a1 = _sc_agg(h, src, dst, zeros)
    out = _final_layer(h, a0, a1,
                       params[f"w1_{i}"], params[f"b1_{i}"].reshape(1, H),
                       params[f"w2_{i}"], params[f"b2_{i}"].reshape(1, H),
                       params[f"gamma_{i}"].reshape(1, H),
                       params[f"beta_{i}"].reshape(1, H),
                       batch3, hw1p, hb1p, hw2p, hb2p)
    return out[:, :OUT]
